# baseline (speedup 1.0000x reference)
import functools

import jax
import jax.numpy as jnp
import numpy as np
from jax import lax
from jax.experimental import pallas as pl
from jax.experimental.pallas import tpu as pltpu
from jax.experimental.pallas import tpu_sc as plsc

N = 10000
NP = 10240
E = 320000
E_TOT = 331776
PAD_E = E_TOT - (E + N)
CHUNK = 96
NCHUNK = E_TOT // (32 * CHUNK)
EPT = NCHUNK * CHUNK
RPT = NP // 16
RB = 1280
GRID = NP // RB

_f32 = jnp.float32
_bf16 = jnp.bfloat16

_SIG = np.empty((128,), np.int32)
for _g in range(4):
    for _k in range(16):
        _SIG[32 * _g + 2 * _k] = 32 * _g + _k
        _SIG[32 * _g + 2 * _k + 1] = 32 * _g + 16 + _k


def _att_tail(j, h, asv_ref, adv_ref, h_ref, s_ref, d_ref, mx_ref):
    h_ref[...] = h.astype(_bf16)
    s = jnp.sum(h * asv_ref[...], axis=1)
    d = jnp.sum(h * adv_ref[...], axis=1)
    s_ref[...] = s[None, :]
    d_ref[...] = d[None, :]
    cur = jnp.concatenate(
        [jnp.full((1, 128), jnp.max(s), _f32), jnp.full((1, 128), jnp.max(d), _f32)], axis=0)

    @pl.when(j == 0)
    def _():
        mx_ref[...] = cur

    @pl.when(j > 0)
    def _():
        mx_ref[...] = jnp.maximum(mx_ref[...], cur)


def _tc0_body(x_ref, w_ref, asv_ref, adv_ref, h_ref, s_ref, d_ref, mx_ref):
    j = pl.program_id(0)
    h = jnp.dot(x_ref[...], w_ref[...], preferred_element_type=_f32)
    _att_tail(j, h, asv_ref, adv_ref, h_ref, s_ref, d_ref, mx_ref)


def _agg(op_ref, dp_ref):
    den = dp_ref[0] + dp_ref[1] + 1e-16
    return (op_ref[0] + op_ref[1]) / den[:, None]


def _tc_comb_body(op_ref, dp_ref, w_ref, asv_ref, adv_ref,
                  h_ref, s_ref, d_ref, mx_ref, *, act):
    j = pl.program_id(0)
    x = _agg(op_ref, dp_ref)
    if act:
        x = jnp.maximum(x, 0.0)
    h = jnp.dot(x, w_ref[...], preferred_element_type=_f32)
    _att_tail(j, h, asv_ref, adv_ref, h_ref, s_ref, d_ref, mx_ref)


def _tc_vae_body(op_ref, dp_ref, wmu_ref, bmu_ref, wlv_ref, blv_ref,
                 w3_ref, asv_ref, adv_ref, eps_ref,
                 mu_ref, lv_ref, z_ref, h_ref, s_ref, d_ref, mx_ref):
    j = pl.program_id(0)
    hmid = _agg(op_ref, dp_ref)
    mu = jnp.dot(hmid, wmu_ref[...], preferred_element_type=_f32) + bmu_ref[...]
    lv = jnp.dot(hmid, wlv_ref[...], preferred_element_type=_f32) + blv_ref[...]
    mu_ref[...] = mu
    lv_ref[...] = lv
    z = eps_ref[...] * jnp.exp(0.5 * lv) + mu
    z_ref[...] = z
    h = jnp.dot(z, w3_ref[...], preferred_element_type=_f32)
    _att_tail(j, h, asv_ref, adv_ref, h_ref, s_ref, d_ref, mx_ref)


def _tc_final_body(op_ref, dp_ref, h_ref):
    h_ref[...] = _agg(op_ref, dp_ref)


def _row_spec(width):
    return pl.BlockSpec((RB, width), lambda j: (j, 0))


_W_SPEC = pl.BlockSpec((128, 128), lambda j: (0, 0))
_VEC_SPEC = pl.BlockSpec((1, 128), lambda j: (0, 0))
_S_SPEC = pl.BlockSpec((1, RB), lambda j: (0, j))
_MX_SPEC = pl.BlockSpec((2, 128), lambda j: (0, 0))
_OP_SPEC = pl.BlockSpec((2, RB, 128), lambda j: (0, j, 0))
_DP_SPEC = pl.BlockSpec((2, RB), lambda j: (0, j))

_ATT_OUT_SHAPE = [
    jax.ShapeDtypeStruct((NP, 128), _bf16),
    jax.ShapeDtypeStruct((1, NP), _f32),
    jax.ShapeDtypeStruct((1, NP), _f32),
    jax.ShapeDtypeStruct((2, 128), _f32),
]
_ATT_OUT_SPECS = [_row_spec(128), _S_SPEC, _S_SPEC, _MX_SPEC]

_tc0 = pl.pallas_call(
    _tc0_body,
    grid=(GRID,),
    in_specs=[_row_spec(128), _W_SPEC, _VEC_SPEC, _VEC_SPEC],
    out_specs=_ATT_OUT_SPECS,
    out_shape=_ATT_OUT_SHAPE,
)


def _tc_comb(act):
    return pl.pallas_call(
        functools.partial(_tc_comb_body, act=act),
        grid=(GRID,),
        in_specs=[_OP_SPEC, _DP_SPEC, _W_SPEC, _VEC_SPEC, _VEC_SPEC],
        out_specs=_ATT_OUT_SPECS,
        out_shape=_ATT_OUT_SHAPE,
    )


_W64_SPEC = pl.BlockSpec((128, 64), lambda j: (0, 0))
_B64_SPEC = pl.BlockSpec((1, 64), lambda j: (0, 0))
_W3_SPEC = pl.BlockSpec((64, 128), lambda j: (0, 0))

_tc_vae = pl.pallas_call(
    _tc_vae_body,
    grid=(GRID,),
    in_specs=[_OP_SPEC, _DP_SPEC, _W64_SPEC, _B64_SPEC, _W64_SPEC, _B64_SPEC,
              _W3_SPEC, _VEC_SPEC, _VEC_SPEC, _row_spec(64)],
    out_specs=[_row_spec(64), _row_spec(64), _row_spec(64)] + _ATT_OUT_SPECS,
    out_shape=[
        jax.ShapeDtypeStruct((NP, 64), _f32),
        jax.ShapeDtypeStruct((NP, 64), _f32),
        jax.ShapeDtypeStruct((NP, 64), _f32),
    ] + _ATT_OUT_SHAPE,
)

_tc_final = pl.pallas_call(
    _tc_final_body,
    grid=(GRID,),
    in_specs=[_OP_SPEC, _DP_SPEC],
    out_specs=[_row_spec(128)],
    out_shape=[jax.ShapeDtypeStruct((NP, 128), _f32)],
)


@functools.cache
def _get_sc_edge():
  mesh = plsc.VectorSubcoreMesh(core_axis_name="c", subcore_axis_name="s")

  @functools.partial(
    pl.kernel,
    mesh=mesh,
    compiler_params=pltpu.CompilerParams(needs_layout_passes=False,
                                         use_tc_tiling_on_sc=False),
    out_type=[
        jax.ShapeDtypeStruct((2, NP, 128), _f32),
        jax.ShapeDtypeStruct((2, NP), _f32),
    ],
    scratch_types=[
        pltpu.VMEM((NP,), _f32),
        pltpu.VMEM((NP,), _f32),
        pltpu.VMEM((1, CHUNK), jnp.int32),
        pltpu.VMEM((1, CHUNK), jnp.int32),
        pltpu.VMEM((1, CHUNK), jnp.int32),
        pltpu.VMEM((1, CHUNK), jnp.int32),
        pltpu.VMEM((CHUNK, 128), _bf16),
        pltpu.VMEM((CHUNK, 128), _bf16),
        pltpu.VMEM((CHUNK, 128), _f32),
        pltpu.VMEM((CHUNK + 16,), _f32),
        pltpu.VMEM((CHUNK + 16,), _f32),
        pltpu.VMEM((16,), _f32),
        pltpu.VMEM_SHARED((NP, 128), _f32),
        pltpu.VMEM_SHARED((NP,), _f32),
        pltpu.SemaphoreType.DMA,
        pltpu.SemaphoreType.DMA,
        pltpu.SemaphoreType.DMA,
        pltpu.SemaphoreType.DMA,
        pltpu.SemaphoreType.DMA,
        pltpu.SemaphoreType.DMA,
    ],
)
  def _sc_edge(h_hbm, s_hbm, d_hbm, src_hbm, dst_hbm,
               g_hbm, out_hbm, den_hbm,
               as_v, ad_v, sidx0, didx0, sidx1, didx1, rows0, rows1, rowsf,
               p0, p1, g_v, acc_sh, den_sh,
               sem_i0, sem_i1, sem_r0, sem_r1, sem_s0, sem_s1):
    c = lax.axis_index("c")
    s = lax.axis_index("s")
    wid = c * 16 + s
    cbase = wid * NCHUNK
    bufs = ((sidx0, didx0, rows0, p0, sem_i0, sem_r0, sem_s0),
            (sidx1, didx1, rows1, p1, sem_i1, sem_r1, sem_s1))

    zero16 = jnp.zeros((16,), _f32)

    def zrow(i, carry):
        for g in range(8):
            rowsf[i, pl.ds(g * 16, 16)] = zero16
        return carry

    lax.fori_loop(0, CHUNK, zrow, 0)

    def zden(i, carry):
        as_v[pl.ds(i * 16, 16)] = zero16
        return carry

    lax.fori_loop(0, RPT // 16, zden, 0)

    base = s * RPT
    nfull = RPT // CHUNK
    for k in range(nfull):
        pltpu.sync_copy(rowsf, acc_sh.at[pl.ds(base + k * CHUNK, CHUNK)])
    rem = RPT - nfull * CHUNK
    if rem:
        pltpu.sync_copy(rowsf.at[pl.ds(0, rem)],
                        acc_sh.at[pl.ds(base + nfull * CHUNK, rem)])
    pltpu.sync_copy(as_v.at[pl.ds(0, RPT)], den_sh.at[pl.ds(base, RPT)])

    pltpu.sync_copy(s_hbm, as_v)
    pltpu.sync_copy(d_hbm, ad_v)
    pltpu.sync_copy(g_hbm, g_v)

    plsc.subcore_barrier()
    gshift = g_v[...][0]

    pltpu.async_copy(src_hbm.at[cbase], sidx0, sem_i0)
    pltpu.async_copy(dst_hbm.at[cbase], didx0, sem_i0)
    pltpu.async_copy(src_hbm.at[cbase + 1], sidx1, sem_i1)
    pltpu.async_copy(dst_hbm.at[cbase + 1], didx1, sem_i1)
    pltpu.make_async_copy(src_hbm.at[cbase], sidx0, sem_i0).wait()
    pltpu.make_async_copy(dst_hbm.at[cbase], didx0, sem_i0).wait()
    pltpu.async_copy(h_hbm.at[sidx0.at[0]], rows0, sem_r0)

    def pair_body(k, carry):
        for b in range(2):
            j = 2 * k + b
            sidx, didx, rows, pv, sem_i, sem_r, sem_s = bufs[b]
            sidx_o, didx_o, rows_o, pv_o, sem_i_o, sem_r_o, sem_s_o = bufs[1 - b]

            @pl.when(j + 1 < NCHUNK)
            def _():
                pltpu.make_async_copy(src_hbm.at[cbase + j + 1], sidx_o,
                                      sem_i_o).wait()
                pltpu.make_async_copy(dst_hbm.at[cbase + j + 1], didx_o,
                                      sem_i_o).wait()
                pltpu.async_copy(h_hbm.at[sidx_o.at[0]], rows_o, sem_r_o)

            pltpu.make_async_copy(h_hbm.at[sidx.at[0]], rows, sem_r).wait()
            for g in range(CHUNK // 16):
                sl = pl.ds(g * 16, 16)
                si = sidx[0, sl]
                di = didx[0, sl]
                e = plsc.load_gather(as_v, [si]) + plsc.load_gather(ad_v, [di])
                e = jnp.where(e >= 0.0, e, 0.2 * e)
                pv[sl] = jnp.exp(e - gshift)

            def scale_row(i, carry2):
                pi = pv[pl.ds(i, 16)][0]
                for g in range(4):
                    w = rows[i, pl.ds(g * 32, 32)]
                    lo, hi = plsc.unpack(w, format=plsc.PackFormat.INTERLEAVED)
                    rowsf[i, pl.ds(g * 32, 16)] = lo * pi
                    rowsf[i, pl.ds(g * 32 + 16, 16)] = hi * pi
                return carry2

            lax.fori_loop(0, CHUNK, scale_row, 0, unroll=8)
            d1 = pltpu.async_copy(rowsf, acc_sh.at[didx.at[0]], sem_s, add=True)
            d2 = pltpu.async_copy(pv.at[pl.ds(0, CHUNK)], den_sh.at[didx.at[0]],
                                  sem_s, add=True)
            d1.wait()
            d2.wait()

            @pl.when(j + 2 < NCHUNK)
            def _():
                pltpu.async_copy(src_hbm.at[cbase + j + 2], sidx, sem_i)
                pltpu.async_copy(dst_hbm.at[cbase + j + 2], didx, sem_i)
        return carry

    lax.fori_loop(0, NCHUNK // 2, pair_body, 0)

    plsc.subcore_barrier()
    pltpu.sync_copy(acc_sh.at[pl.ds(base, RPT)], out_hbm.at[c, pl.ds(base, RPT)])
    pltpu.sync_copy(den_sh.at[pl.ds(base, RPT)], den_hbm.at[c, pl.ds(base, RPT)])

  return _sc_edge


def _shift16(mx):
    m = mx[0, 0] + mx[1, 0]
    g = jnp.where(m >= 0.0, m, 0.2 * m)
    return jnp.full((16,), g, _f32)


def kernel(x, edge_index, W1, as1, ad1, W2, as2, ad2, Wmu, bmu, Wlv, blv,
           W3, as3, ad3, W4, as4, ad4):
    xp = jnp.zeros((NP, 128), _f32).at[:N].set(x)
    loop = jnp.arange(N, dtype=jnp.int32)
    pad_src = jnp.arange(PAD_E, dtype=jnp.int32) % N
    pad_dst = N + (jnp.arange(PAD_E, dtype=jnp.int32) % (NP - N))
    src = jnp.concatenate([edge_index[0].astype(jnp.int32), loop, pad_src])
    dst = jnp.concatenate([edge_index[1].astype(jnp.int32), loop, pad_dst])
    srcm = src.reshape(-1, 1, CHUNK)
    dstm = dst.reshape(-1, 1, CHUNK)

    def v2(a):
        return a[_SIG].reshape(1, -1)

    def wsig(w):
        return w[:, _SIG]

    h1, s1, d1, mx1 = _tc0(xp, wsig(W1), v2(as1), v2(ad1))
    op1, dp1 = _get_sc_edge()(h1, s1[0], d1[0], srcm, dstm, _shift16(mx1))

    h2, s2, d2, mx2 = _tc_comb(True)(op1, dp1, wsig(W2), v2(as2), v2(ad2))
    op2, dp2 = _get_sc_edge()(h2, s2[0], d2[0], srcm, dstm, _shift16(mx2))

    eps = jnp.zeros((NP, 64), _f32).at[:N].set(
        jax.random.normal(jax.random.key(42), (N, 64), _f32))
    mu, lv, z, h3, s3, d3, mx3 = _tc_vae(
        op2, dp2, Wmu, bmu.reshape(1, -1), Wlv, blv.reshape(1, -1),
        wsig(W3), v2(as3), v2(ad3), eps)
    op3, dp3 = _get_sc_edge()(h3, s3[0], d3[0], srcm, dstm, _shift16(mx3))

    h4, s4, d4, mx4 = _tc_comb(True)(op3, dp3, wsig(W4), v2(as4), v2(ad4))
    op4, dp4 = _get_sc_edge()(h4, s4[0], d4[0], srcm, dstm, _shift16(mx4))

    (out,) = _tc_final(op4, dp4)
    return (out[:N], mu[:N], lv[:N], z[:N])


# speedup vs baseline: 1.5696x; 1.5696x over previous
import functools

import jax
import jax.numpy as jnp
import numpy as np
from jax import lax
from jax.experimental import pallas as pl
from jax.experimental.pallas import tpu as pltpu
from jax.experimental.pallas import tpu_sc as plsc

N = 10000
NP = 10240
E = 320000
E_TOT = 331776
PAD_E = E_TOT - (E + N)
CHUNK = 96
NCHUNK = E_TOT // (32 * CHUNK)
EPT = NCHUNK * CHUNK
RPT = NP // 16
RB = 1280
GRID = NP // RB

_f32 = jnp.float32
_bf16 = jnp.bfloat16

_SIG = np.empty((128,), np.int32)
for _g in range(4):
    for _k in range(16):
        _SIG[32 * _g + 2 * _k] = 32 * _g + _k
        _SIG[32 * _g + 2 * _k + 1] = 32 * _g + 16 + _k


def _att_tail(j, h, asv_ref, adv_ref, h_ref, s_ref, d_ref, mx_ref):
    h_ref[...] = h
    s = jnp.sum(h * asv_ref[...], axis=1)
    d = jnp.sum(h * adv_ref[...], axis=1)
    s_ref[...] = s[None, :]
    d_ref[...] = d[None, :]
    cur = jnp.concatenate(
        [jnp.full((1, 128), jnp.max(s), _f32), jnp.full((1, 128), jnp.max(d), _f32)], axis=0)

    @pl.when(j == 0)
    def _():
        mx_ref[...] = cur

    @pl.when(j > 0)
    def _():
        mx_ref[...] = jnp.maximum(mx_ref[...], cur)


def _tc0_body(x_ref, w_ref, asv_ref, adv_ref, h_ref, s_ref, d_ref, mx_ref):
    j = pl.program_id(0)
    h = jnp.dot(x_ref[...], w_ref[...], preferred_element_type=_f32)
    _att_tail(j, h, asv_ref, adv_ref, h_ref, s_ref, d_ref, mx_ref)


def _agg(op_ref, dp_ref):
    den = dp_ref[0] + dp_ref[1] + 1e-16
    return (op_ref[0] + op_ref[1]) / den[:, None]


def _tc_comb_body(op_ref, dp_ref, w_ref, asv_ref, adv_ref,
                  h_ref, s_ref, d_ref, mx_ref, *, act):
    j = pl.program_id(0)
    x = _agg(op_ref, dp_ref)
    if act:
        x = jnp.maximum(x, 0.0)
    h = jnp.dot(x, w_ref[...], preferred_element_type=_f32)
    _att_tail(j, h, asv_ref, adv_ref, h_ref, s_ref, d_ref, mx_ref)


def _tc_vae_body(op_ref, dp_ref, wmu_ref, bmu_ref, wlv_ref, blv_ref,
                 w3_ref, asv_ref, adv_ref, eps_ref,
                 mu_ref, lv_ref, z_ref, h_ref, s_ref, d_ref, mx_ref):
    j = pl.program_id(0)
    hmid = _agg(op_ref, dp_ref)
    mu = jnp.dot(hmid, wmu_ref[...], preferred_element_type=_f32) + bmu_ref[...]
    lv = jnp.dot(hmid, wlv_ref[...], preferred_element_type=_f32) + blv_ref[...]
    mu_ref[...] = mu
    lv_ref[...] = lv
    z = eps_ref[...] * jnp.exp(0.5 * lv) + mu
    z_ref[...] = z
    h = jnp.dot(z, w3_ref[...], preferred_element_type=_f32)
    _att_tail(j, h, asv_ref, adv_ref, h_ref, s_ref, d_ref, mx_ref)


def _tc_final_body(op_ref, dp_ref, h_ref):
    h_ref[...] = _agg(op_ref, dp_ref)


def _row_spec(width):
    return pl.BlockSpec((RB, width), lambda j: (j, 0))


_W_SPEC = pl.BlockSpec((128, 128), lambda j: (0, 0))
_VEC_SPEC = pl.BlockSpec((1, 128), lambda j: (0, 0))
_S_SPEC = pl.BlockSpec((1, RB), lambda j: (0, j))
_MX_SPEC = pl.BlockSpec((2, 128), lambda j: (0, 0))
_OP_SPEC = pl.BlockSpec((2, RB, 128), lambda j: (0, j, 0))
_DP_SPEC = pl.BlockSpec((2, RB), lambda j: (0, j))

_ATT_OUT_SHAPE = [
    jax.ShapeDtypeStruct((NP, 128), _f32),
    jax.ShapeDtypeStruct((1, NP), _f32),
    jax.ShapeDtypeStruct((1, NP), _f32),
    jax.ShapeDtypeStruct((2, 128), _f32),
]
_ATT_OUT_SPECS = [_row_spec(128), _S_SPEC, _S_SPEC, _MX_SPEC]

_tc0 = pl.pallas_call(
    _tc0_body,
    grid=(GRID,),
    in_specs=[_row_spec(128), _W_SPEC, _VEC_SPEC, _VEC_SPEC],
    out_specs=_ATT_OUT_SPECS,
    out_shape=_ATT_OUT_SHAPE,
)


def _tc_comb(act):
    return pl.pallas_call(
        functools.partial(_tc_comb_body, act=act),
        grid=(GRID,),
        in_specs=[_OP_SPEC, _DP_SPEC, _W_SPEC, _VEC_SPEC, _VEC_SPEC],
        out_specs=_ATT_OUT_SPECS,
        out_shape=_ATT_OUT_SHAPE,
    )


_W64_SPEC = pl.BlockSpec((128, 64), lambda j: (0, 0))
_B64_SPEC = pl.BlockSpec((1, 64), lambda j: (0, 0))
_W3_SPEC = pl.BlockSpec((64, 128), lambda j: (0, 0))

_tc_vae = pl.pallas_call(
    _tc_vae_body,
    grid=(GRID,),
    in_specs=[_OP_SPEC, _DP_SPEC, _W64_SPEC, _B64_SPEC, _W64_SPEC, _B64_SPEC,
              _W3_SPEC, _VEC_SPEC, _VEC_SPEC, _row_spec(64)],
    out_specs=[_row_spec(64), _row_spec(64), _row_spec(64)] + _ATT_OUT_SPECS,
    out_shape=[
        jax.ShapeDtypeStruct((NP, 64), _f32),
        jax.ShapeDtypeStruct((NP, 64), _f32),
        jax.ShapeDtypeStruct((NP, 64), _f32),
    ] + _ATT_OUT_SHAPE,
)

_tc_final = pl.pallas_call(
    _tc_final_body,
    grid=(GRID,),
    in_specs=[_OP_SPEC, _DP_SPEC],
    out_specs=[_row_spec(128)],
    out_shape=[jax.ShapeDtypeStruct((NP, 128), _f32)],
)


@functools.cache
def _get_sc_edge():
  mesh = plsc.VectorSubcoreMesh(core_axis_name="c", subcore_axis_name="s")

  @functools.partial(
    pl.kernel,
    mesh=mesh,
    compiler_params=pltpu.CompilerParams(needs_layout_passes=False),
    out_type=[
        jax.ShapeDtypeStruct((2, NP, 128), _f32),
        jax.ShapeDtypeStruct((2, NP), _f32),
    ],
    scratch_types=[
        pltpu.VMEM((NP,), _f32),
        pltpu.VMEM((NP,), _f32),
        pltpu.VMEM((1, CHUNK), jnp.int32),
        pltpu.VMEM((1, CHUNK), jnp.int32),
        pltpu.VMEM((1, CHUNK), jnp.int32),
        pltpu.VMEM((1, CHUNK), jnp.int32),
        pltpu.VMEM((CHUNK, 128), _f32),
        pltpu.VMEM((CHUNK, 128), _f32),
        pltpu.VMEM((CHUNK + 16,), _f32),
        pltpu.VMEM((CHUNK + 16,), _f32),
        pltpu.VMEM((16,), _f32),
        pltpu.VMEM_SHARED((NP, 128), _f32),
        pltpu.VMEM_SHARED((NP,), _f32),
        pltpu.SemaphoreType.DMA,
        pltpu.SemaphoreType.DMA,
        pltpu.SemaphoreType.DMA,
        pltpu.SemaphoreType.DMA,
        pltpu.SemaphoreType.DMA,
        pltpu.SemaphoreType.DMA,
    ],
)
  def _sc_edge(h_hbm, s_hbm, d_hbm, src_hbm, dst_hbm,
               g_hbm, out_hbm, den_hbm,
               as_v, ad_v, sidx0, didx0, sidx1, didx1, rows0, rows1,
               p0, p1, g_v, acc_sh, den_sh,
               sem_i0, sem_i1, sem_r0, sem_r1, sem_s0, sem_s1):
    c = lax.axis_index("c")
    s = lax.axis_index("s")
    wid = c * 16 + s
    cbase = wid * NCHUNK
    bufs = ((sidx0, didx0, rows0, p0, sem_i0, sem_r0, sem_s0),
            (sidx1, didx1, rows1, p1, sem_i1, sem_r1, sem_s1))

    zero16 = jnp.zeros((16,), _f32)

    def zrow(i, carry):
        for g in range(8):
            rows0[i, pl.ds(g * 16, 16)] = zero16
        return carry

    lax.fori_loop(0, CHUNK, zrow, 0)

    def zden(i, carry):
        as_v[pl.ds(i * 16, 16)] = zero16
        return carry

    lax.fori_loop(0, RPT // 16, zden, 0)

    base = s * RPT
    nfull = RPT // CHUNK
    for k in range(nfull):
        pltpu.sync_copy(rows0, acc_sh.at[pl.ds(base + k * CHUNK, CHUNK)])
    rem = RPT - nfull * CHUNK
    if rem:
        pltpu.sync_copy(rows0.at[pl.ds(0, rem)],
                        acc_sh.at[pl.ds(base + nfull * CHUNK, rem)])
    pltpu.sync_copy(as_v.at[pl.ds(0, RPT)], den_sh.at[pl.ds(base, RPT)])

    pltpu.sync_copy(s_hbm, as_v)
    pltpu.sync_copy(d_hbm, ad_v)
    pltpu.sync_copy(g_hbm, g_v)

    plsc.subcore_barrier()
    gshift = g_v[...][0]

    pltpu.async_copy(src_hbm.at[cbase], sidx0, sem_i0)
    pltpu.async_copy(dst_hbm.at[cbase], didx0, sem_i0)
    pltpu.async_copy(src_hbm.at[cbase + 1], sidx1, sem_i1)
    pltpu.async_copy(dst_hbm.at[cbase + 1], didx1, sem_i1)
    pltpu.make_async_copy(src_hbm.at[cbase], sidx0, sem_i0).wait()
    pltpu.make_async_copy(dst_hbm.at[cbase], didx0, sem_i0).wait()
    pltpu.async_copy(h_hbm.at[sidx0.at[0]], rows0, sem_r0)

    def pair_body(k, carry):
        for b in range(2):
            j = 2 * k + b
            sidx, didx, rows, pv, sem_i, sem_r, sem_s = bufs[b]
            sidx_o, didx_o, rows_o, pv_o, sem_i_o, sem_r_o, sem_s_o = bufs[1 - b]

            @pl.when(j + 1 < NCHUNK)
            def _():
                pltpu.make_async_copy(src_hbm.at[cbase + j + 1], sidx_o,
                                      sem_i_o).wait()
                pltpu.make_async_copy(dst_hbm.at[cbase + j + 1], didx_o,
                                      sem_i_o).wait()
                pltpu.async_copy(h_hbm.at[sidx_o.at[0]], rows_o, sem_r_o)

            pltpu.make_async_copy(h_hbm.at[sidx.at[0]], rows, sem_r).wait()
            for g in range(CHUNK // 16):
                sl = pl.ds(g * 16, 16)
                si = sidx[0, sl]
                di = didx[0, sl]
                e = plsc.load_gather(as_v, [si]) + plsc.load_gather(ad_v, [di])
                e = jnp.where(e >= 0.0, e, 0.2 * e)
                pv[sl] = jnp.exp(e - gshift)

            def scale_row(i, carry2):
                pi = pv[pl.ds(i, 16)][0]
                for g in range(8):
                    sl = pl.ds(g * 16, 16)
                    rows[i, sl] = rows[i, sl] * pi
                return carry2

            lax.fori_loop(0, CHUNK, scale_row, 0, unroll=8)
            d1 = pltpu.async_copy(rows, acc_sh.at[didx.at[0]], sem_s, add=True)
            d2 = pltpu.async_copy(pv.at[pl.ds(0, CHUNK)], den_sh.at[didx.at[0]],
                                  sem_s, add=True)
            d1.wait()
            d2.wait()

            @pl.when(j + 2 < NCHUNK)
            def _():
                pltpu.async_copy(src_hbm.at[cbase + j + 2], sidx, sem_i)
                pltpu.async_copy(dst_hbm.at[cbase + j + 2], didx, sem_i)
        return carry

    lax.fori_loop(0, NCHUNK // 2, pair_body, 0)

    plsc.subcore_barrier()
    pltpu.sync_copy(acc_sh.at[pl.ds(base, RPT)], out_hbm.at[c, pl.ds(base, RPT)])
    pltpu.sync_copy(den_sh.at[pl.ds(base, RPT)], den_hbm.at[c, pl.ds(base, RPT)])

  return _sc_edge


def _shift16(mx):
    m = mx[0, 0] + mx[1, 0]
    g = jnp.where(m >= 0.0, m, 0.2 * m)
    return jnp.full((16,), g, _f32)


def kernel(x, edge_index, W1, as1, ad1, W2, as2, ad2, Wmu, bmu, Wlv, blv,
           W3, as3, ad3, W4, as4, ad4):
    xp = jnp.zeros((NP, 128), _f32).at[:N].set(x)
    loop = jnp.arange(N, dtype=jnp.int32)
    pad_src = jnp.arange(PAD_E, dtype=jnp.int32) % N
    pad_dst = N + (jnp.arange(PAD_E, dtype=jnp.int32) % (NP - N))
    src = jnp.concatenate([edge_index[0].astype(jnp.int32), loop, pad_src])
    dst = jnp.concatenate([edge_index[1].astype(jnp.int32), loop, pad_dst])
    srcm = src.reshape(-1, 1, CHUNK)
    dstm = dst.reshape(-1, 1, CHUNK)

    def v2(a):
        return a.reshape(1, -1)

    def wsig(w):
        return w

    h1, s1, d1, mx1 = _tc0(xp, wsig(W1), v2(as1), v2(ad1))
    op1, dp1 = _get_sc_edge()(h1, s1[0], d1[0], srcm, dstm, _shift16(mx1))

    h2, s2, d2, mx2 = _tc_comb(True)(op1, dp1, wsig(W2), v2(as2), v2(ad2))
    op2, dp2 = _get_sc_edge()(h2, s2[0], d2[0], srcm, dstm, _shift16(mx2))

    eps = jnp.zeros((NP, 64), _f32).at[:N].set(
        jax.random.normal(jax.random.key(42), (N, 64), _f32))
    mu, lv, z, h3, s3, d3, mx3 = _tc_vae(
        op2, dp2, Wmu, bmu.reshape(1, -1), Wlv, blv.reshape(1, -1),
        wsig(W3), v2(as3), v2(ad3), eps)
    op3, dp3 = _get_sc_edge()(h3, s3[0], d3[0], srcm, dstm, _shift16(mx3))

    h4, s4, d4, mx4 = _tc_comb(True)(op3, dp3, wsig(W4), v2(as4), v2(ad4))
    op4, dp4 = _get_sc_edge()(h4, s4[0], d4[0], srcm, dstm, _shift16(mx4))

    (out,) = _tc_final(op4, dp4)
    return (out[:N], mu[:N], lv[:N], z[:N])


# speedup vs baseline: 1.5697x; 1.0001x over previous
import functools

import jax
import jax.numpy as jnp
from jax import lax
from jax.experimental import pallas as pl
from jax.experimental.pallas import tpu as pltpu
from jax.experimental.pallas import tpu_sc as plsc

N = 10000
NP = 10240
E = 320000
E_TOT = 331776
PAD_E = E_TOT - (E + N)
CHUNK = 96
NCHUNK = E_TOT // (32 * CHUNK)
EPT = NCHUNK * CHUNK
RPT = NP // 16
RB = 1280
GRID = NP // RB

_f32 = jnp.float32


def _att_tail(j, h, asv_ref, adv_ref, h_ref, s_ref, d_ref, mx_ref):
    h_ref[...] = h
    s = jnp.sum(h * asv_ref[...], axis=1)
    d = jnp.sum(h * adv_ref[...], axis=1)
    s_ref[...] = s[None, :]
    d_ref[...] = d[None, :]
    cur = jnp.concatenate(
        [jnp.full((1, 128), jnp.max(s), _f32), jnp.full((1, 128), jnp.max(d), _f32)], axis=0)

    @pl.when(j == 0)
    def _():
        mx_ref[...] = cur

    @pl.when(j > 0)
    def _():
        mx_ref[...] = jnp.maximum(mx_ref[...], cur)


def _tc0_body(x_ref, w_ref, asv_ref, adv_ref, h_ref, s_ref, d_ref, mx_ref):
    j = pl.program_id(0)
    h = jnp.dot(x_ref[...], w_ref[...], preferred_element_type=_f32)
    _att_tail(j, h, asv_ref, adv_ref, h_ref, s_ref, d_ref, mx_ref)


def _agg(op_ref, dp_ref):
    den = dp_ref[0] + dp_ref[1] + 1e-16
    return (op_ref[0] + op_ref[1]) / den[:, None]


def _tc_comb_body(op_ref, dp_ref, w_ref, asv_ref, adv_ref,
                  h_ref, s_ref, d_ref, mx_ref, *, act):
    j = pl.program_id(0)
    x = _agg(op_ref, dp_ref)
    if act:
        x = jnp.maximum(x, 0.0)
    h = jnp.dot(x, w_ref[...], preferred_element_type=_f32)
    _att_tail(j, h, asv_ref, adv_ref, h_ref, s_ref, d_ref, mx_ref)


def _tc_vae_body(op_ref, dp_ref, wmu_ref, bmu_ref, wlv_ref, blv_ref,
                 w3_ref, asv_ref, adv_ref, eps_ref,
                 mu_ref, lv_ref, z_ref, h_ref, s_ref, d_ref, mx_ref):
    j = pl.program_id(0)
    hmid = _agg(op_ref, dp_ref)
    mu = jnp.dot(hmid, wmu_ref[...], preferred_element_type=_f32) + bmu_ref[...]
    lv = jnp.dot(hmid, wlv_ref[...], preferred_element_type=_f32) + blv_ref[...]
    mu_ref[...] = mu
    lv_ref[...] = lv
    z = eps_ref[...] * jnp.exp(0.5 * lv) + mu
    z_ref[...] = z
    h = jnp.dot(z, w3_ref[...], preferred_element_type=_f32)
    _att_tail(j, h, asv_ref, adv_ref, h_ref, s_ref, d_ref, mx_ref)


def _tc_final_body(op_ref, dp_ref, h_ref):
    h_ref[...] = _agg(op_ref, dp_ref)


def _row_spec(width):
    return pl.BlockSpec((RB, width), lambda j: (j, 0))


_W_SPEC = pl.BlockSpec((128, 128), lambda j: (0, 0))
_VEC_SPEC = pl.BlockSpec((1, 128), lambda j: (0, 0))
_S_SPEC = pl.BlockSpec((1, RB), lambda j: (0, j))
_MX_SPEC = pl.BlockSpec((2, 128), lambda j: (0, 0))
_OP_SPEC = pl.BlockSpec((2, RB, 128), lambda j: (0, j, 0))
_DP_SPEC = pl.BlockSpec((2, RB), lambda j: (0, j))

_ATT_OUT_SHAPE = [
    jax.ShapeDtypeStruct((NP, 128), _f32),
    jax.ShapeDtypeStruct((1, NP), _f32),
    jax.ShapeDtypeStruct((1, NP), _f32),
    jax.ShapeDtypeStruct((2, 128), _f32),
]
_ATT_OUT_SPECS = [_row_spec(128), _S_SPEC, _S_SPEC, _MX_SPEC]

_tc0 = pl.pallas_call(
    _tc0_body,
    grid=(GRID,),
    in_specs=[_row_spec(128), _W_SPEC, _VEC_SPEC, _VEC_SPEC],
    out_specs=_ATT_OUT_SPECS,
    out_shape=_ATT_OUT_SHAPE,
)


def _tc_comb(act):
    return pl.pallas_call(
        functools.partial(_tc_comb_body, act=act),
        grid=(GRID,),
        in_specs=[_OP_SPEC, _DP_SPEC, _W_SPEC, _VEC_SPEC, _VEC_SPEC],
        out_specs=_ATT_OUT_SPECS,
        out_shape=_ATT_OUT_SHAPE,
    )


_W64_SPEC = pl.BlockSpec((128, 64), lambda j: (0, 0))
_B64_SPEC = pl.BlockSpec((1, 64), lambda j: (0, 0))
_W3_SPEC = pl.BlockSpec((64, 128), lambda j: (0, 0))

_tc_vae = pl.pallas_call(
    _tc_vae_body,
    grid=(GRID,),
    in_specs=[_OP_SPEC, _DP_SPEC, _W64_SPEC, _B64_SPEC, _W64_SPEC, _B64_SPEC,
              _W3_SPEC, _VEC_SPEC, _VEC_SPEC, _row_spec(64)],
    out_specs=[_row_spec(64), _row_spec(64), _row_spec(64)] + _ATT_OUT_SPECS,
    out_shape=[
        jax.ShapeDtypeStruct((NP, 64), _f32),
        jax.ShapeDtypeStruct((NP, 64), _f32),
        jax.ShapeDtypeStruct((NP, 64), _f32),
    ] + _ATT_OUT_SHAPE,
)

_tc_final = pl.pallas_call(
    _tc_final_body,
    grid=(GRID,),
    in_specs=[_OP_SPEC, _DP_SPEC],
    out_specs=[_row_spec(128)],
    out_shape=[jax.ShapeDtypeStruct((NP, 128), _f32)],
)


@functools.cache
def _get_sc_edge():
  mesh = plsc.VectorSubcoreMesh(core_axis_name="c", subcore_axis_name="s")

  @functools.partial(
    pl.kernel,
    mesh=mesh,
    compiler_params=pltpu.CompilerParams(needs_layout_passes=False),
    out_type=[
        jax.ShapeDtypeStruct((2, NP, 128), _f32),
        jax.ShapeDtypeStruct((2, NP), _f32),
    ],
    scratch_types=[
        pltpu.VMEM((NP,), _f32),
        pltpu.VMEM((NP,), _f32),
        pltpu.VMEM((1, CHUNK), jnp.int32),
        pltpu.VMEM((1, CHUNK), jnp.int32),
        pltpu.VMEM((1, CHUNK), jnp.int32),
        pltpu.VMEM((1, CHUNK), jnp.int32),
        pltpu.VMEM((CHUNK, 128), _f32),
        pltpu.VMEM((CHUNK, 128), _f32),
        pltpu.VMEM((CHUNK + 16,), _f32),
        pltpu.VMEM((CHUNK + 16,), _f32),
        pltpu.VMEM((16,), _f32),
        pltpu.VMEM_SHARED((NP, 128), _f32),
        pltpu.VMEM_SHARED((NP,), _f32),
        pltpu.SemaphoreType.DMA,
        pltpu.SemaphoreType.DMA,
        pltpu.SemaphoreType.DMA,
        pltpu.SemaphoreType.DMA,
        pltpu.SemaphoreType.DMA,
        pltpu.SemaphoreType.DMA,
    ],
)
  def _sc_edge(h_hbm, s_hbm, d_hbm, src_hbm, dst_hbm,
               g_hbm, out_hbm, den_hbm,
               as_v, ad_v, sidx0, didx0, sidx1, didx1, rows0, rows1,
               p0, p1, g_v, acc_sh, den_sh,
               sem_i0, sem_i1, sem_r0, sem_r1, sem_s0, sem_s1):
    c = lax.axis_index("c")
    s = lax.axis_index("s")
    wid = c * 16 + s
    cbase = wid * NCHUNK
    bufs = ((sidx0, didx0, rows0, p0, sem_i0, sem_r0, sem_s0),
            (sidx1, didx1, rows1, p1, sem_i1, sem_r1, sem_s1))

    zero16 = jnp.zeros((16,), _f32)

    def zrow(i, carry):
        for g in range(8):
            rows0[i, pl.ds(g * 16, 16)] = zero16
        return carry

    lax.fori_loop(0, CHUNK, zrow, 0)

    def zden(i, carry):
        as_v[pl.ds(i * 16, 16)] = zero16
        return carry

    lax.fori_loop(0, RPT // 16, zden, 0)

    base = s * RPT
    nfull = RPT // CHUNK
    for k in range(nfull):
        pltpu.sync_copy(rows0, acc_sh.at[pl.ds(base + k * CHUNK, CHUNK)])
    rem = RPT - nfull * CHUNK
    if rem:
        pltpu.sync_copy(rows0.at[pl.ds(0, rem)],
                        acc_sh.at[pl.ds(base + nfull * CHUNK, rem)])
    pltpu.sync_copy(as_v.at[pl.ds(0, RPT)], den_sh.at[pl.ds(base, RPT)])

    pltpu.sync_copy(s_hbm, as_v)
    pltpu.sync_copy(d_hbm, ad_v)
    pltpu.sync_copy(g_hbm, g_v)

    plsc.subcore_barrier()
    gshift = g_v[...][0]

    pltpu.async_copy(src_hbm.at[cbase], sidx0, sem_i0)
    pltpu.async_copy(dst_hbm.at[cbase], didx0, sem_i0)
    pltpu.async_copy(src_hbm.at[cbase + 1], sidx1, sem_i1)
    pltpu.async_copy(dst_hbm.at[cbase + 1], didx1, sem_i1)
    pltpu.make_async_copy(src_hbm.at[cbase], sidx0, sem_i0).wait()
    pltpu.make_async_copy(dst_hbm.at[cbase], didx0, sem_i0).wait()
    pltpu.async_copy(h_hbm.at[sidx0.at[0]], rows0, sem_r0)

    def pair_body(k, carry):
        for b in range(2):
            j = 2 * k + b
            sidx, didx, rows, pv, sem_i, sem_r, sem_s = bufs[b]
            sidx_o, didx_o, rows_o, pv_o, sem_i_o, sem_r_o, sem_s_o = bufs[1 - b]

            @pl.when(j + 1 < NCHUNK)
            def _():
                pltpu.make_async_copy(src_hbm.at[cbase + j + 1], sidx_o,
                                      sem_i_o).wait()
                pltpu.make_async_copy(dst_hbm.at[cbase + j + 1], didx_o,
                                      sem_i_o).wait()
                pltpu.async_copy(h_hbm.at[sidx_o.at[0]], rows_o, sem_r_o)

            for g in range(CHUNK // 16):
                sl = pl.ds(g * 16, 16)
                si = sidx[0, sl]
                di = didx[0, sl]
                e = plsc.load_gather(as_v, [si]) + plsc.load_gather(ad_v, [di])
                e = jnp.where(e >= 0.0, e, 0.2 * e)
                pv[sl] = jnp.exp(e - gshift)

            pltpu.make_async_copy(h_hbm.at[sidx.at[0]], rows, sem_r).wait()

            def scale_row(i, carry2):
                pi = pv[pl.ds(i, 16)][0]
                for g in range(8):
                    sl = pl.ds(g * 16, 16)
                    rows[i, sl] = rows[i, sl] * pi
                return carry2

            lax.fori_loop(0, CHUNK, scale_row, 0, unroll=8)
            d1 = pltpu.async_copy(rows, acc_sh.at[didx.at[0]], sem_s, add=True)
            d2 = pltpu.async_copy(pv.at[pl.ds(0, CHUNK)], den_sh.at[didx.at[0]],
                                  sem_s, add=True)
            d1.wait()
            d2.wait()

            @pl.when(j + 2 < NCHUNK)
            def _():
                pltpu.async_copy(src_hbm.at[cbase + j + 2], sidx, sem_i)
                pltpu.async_copy(dst_hbm.at[cbase + j + 2], didx, sem_i)
        return carry

    lax.fori_loop(0, NCHUNK // 2, pair_body, 0)

    plsc.subcore_barrier()
    pltpu.sync_copy(acc_sh.at[pl.ds(base, RPT)], out_hbm.at[c, pl.ds(base, RPT)])
    pltpu.sync_copy(den_sh.at[pl.ds(base, RPT)], den_hbm.at[c, pl.ds(base, RPT)])

  return _sc_edge


def _shift16(mx):
    m = mx[0, 0] + mx[1, 0]
    g = jnp.where(m >= 0.0, m, 0.2 * m)
    return jnp.full((16,), g, _f32)


def kernel(x, edge_index, W1, as1, ad1, W2, as2, ad2, Wmu, bmu, Wlv, blv,
           W3, as3, ad3, W4, as4, ad4):
    xp = jnp.zeros((NP, 128), _f32).at[:N].set(x)
    loop = jnp.arange(N, dtype=jnp.int32)
    pad_src = jnp.arange(PAD_E, dtype=jnp.int32) % N
    pad_dst = N + (jnp.arange(PAD_E, dtype=jnp.int32) % (NP - N))
    src = jnp.concatenate([edge_index[0].astype(jnp.int32), loop, pad_src])
    dst = jnp.concatenate([edge_index[1].astype(jnp.int32), loop, pad_dst])
    srcm = src.reshape(-1, 1, CHUNK)
    dstm = dst.reshape(-1, 1, CHUNK)

    def v2(a):
        return a.reshape(1, -1)

    h1, s1, d1, mx1 = _tc0(xp, W1, v2(as1), v2(ad1))
    op1, dp1 = _get_sc_edge()(h1, s1[0], d1[0], srcm, dstm, _shift16(mx1))

    h2, s2, d2, mx2 = _tc_comb(True)(op1, dp1, W2, v2(as2), v2(ad2))
    op2, dp2 = _get_sc_edge()(h2, s2[0], d2[0], srcm, dstm, _shift16(mx2))

    eps = jnp.zeros((NP, 64), _f32).at[:N].set(
        jax.random.normal(jax.random.key(42), (N, 64), _f32))
    mu, lv, z, h3, s3, d3, mx3 = _tc_vae(
        op2, dp2, Wmu, bmu.reshape(1, -1), Wlv, blv.reshape(1, -1),
        W3, v2(as3), v2(ad3), eps)
    op3, dp3 = _get_sc_edge()(h3, s3[0], d3[0], srcm, dstm, _shift16(mx3))

    h4, s4, d4, mx4 = _tc_comb(True)(op3, dp3, W4, v2(as4), v2(ad4))
    op4, dp4 = _get_sc_edge()(h4, s4[0], d4[0], srcm, dstm, _shift16(mx4))

    (out,) = _tc_final(op4, dp4)
    return (out[:N], mu[:N], lv[:N], z[:N])
